# baseline (speedup 1.0000x reference)
import functools

import jax
import jax.numpy as jnp
from jax import lax
from jax.experimental import pallas as pl
from jax.experimental.pallas import tpu as pltpu
from jax.experimental.pallas import tpu_sc as plsc

NC = 2
NS = 16
NW = NC * NS

CHUNK = 8
NBUF = 3


@functools.partial(jax.jit, static_argnums=(2, 3))
def _gather_rows(table, idx, n, d):
    b_per_w = n // NW
    n_chunks = b_per_w // CHUNK

    mesh = plsc.VectorSubcoreMesh(core_axis_name="c", subcore_axis_name="s")

    @functools.partial(
        pl.kernel,
        mesh=mesh,
        out_type=jax.ShapeDtypeStruct((n, d), jnp.float32),
        scratch_types=[
            pltpu.VMEM((b_per_w,), jnp.int32),
            [pltpu.VMEM((CHUNK, d), jnp.float32) for _ in range(NBUF)],
            [pltpu.SemaphoreType.DMA for _ in range(NBUF)],
            [pltpu.SemaphoreType.DMA for _ in range(NBUF)],
        ],
    )
    def k(table_hbm, idx_hbm, out_hbm, idx_v, bufs, gsems, ssems):
        wid = lax.axis_index("s") * NC + lax.axis_index("c")
        base = wid * b_per_w
        pltpu.sync_copy(idx_hbm.at[pl.ds(base, b_per_w)], idx_v)

        def start_gather(g, b):
            off = pl.multiple_of(g * CHUNK, CHUNK)
            pltpu.async_copy(
                table_hbm.at[idx_v.at[pl.ds(off, CHUNK)]], bufs[b], gsems[b]
            )

        def start_scatter(g, b):
            off = pl.multiple_of(g * CHUNK, CHUNK)
            pltpu.async_copy(
                bufs[b], out_hbm.at[pl.ds(base + off, CHUNK)], ssems[b]
            )

        def wait_scatter(b):
            pltpu.make_async_copy(
                bufs[b], out_hbm.at[pl.ds(base, CHUNK)], ssems[b]
            ).wait()

        def wait_gather(b):
            pltpu.make_async_copy(
                table_hbm.at[idx_v.at[pl.ds(0, CHUNK)]], bufs[b], gsems[b]
            ).wait()

        n_outer = (n_chunks + NBUF) // NBUF

        def body(p, carry):
            for b in range(NBUF):
                g = p * NBUF + b
                bprev = (b - 1) % NBUF

                @pl.when(jnp.logical_and(g >= NBUF, g < n_chunks))
                def _():
                    wait_scatter(b)

                @pl.when(g < n_chunks)
                def _():
                    start_gather(g, b)

                @pl.when(jnp.logical_and(g >= 1, g <= n_chunks))
                def _():
                    wait_gather(bprev)
                    start_scatter(g - 1, bprev)

            return carry

        lax.fori_loop(0, n_outer, body, 0)

        for b in range(NBUF):
            wait_scatter(b)

    return k(table, idx)


def kernel(input, embed_weight):
    b, t = input.shape
    v, d = embed_weight.shape
    idx = input.reshape(b * t).astype(jnp.int32)
    out = _gather_rows(embed_weight, idx, b * t, d)
    return out.reshape(b, t, d)
